# baseline (speedup 1.0000x reference)
import functools

import jax
import jax.numpy as jnp
from jax import lax
from jax.experimental import pallas as pl
from jax.experimental.pallas import tpu as pltpu
from jax.experimental.pallas import tpu_sc as plsc

B = 16384
L = 50
EMB = 32
NW = 32
BPW = B // NW
EPB = 2
NBUF = 8
NGRP = BPW // EPB
HALF = 16

_mesh = plsc.VectorSubcoreMesh(core_axis_name="c", subcore_axis_name="s")


@functools.partial(
    pl.kernel,
    out_type=jax.ShapeDtypeStruct((B, EMB), jnp.float32),
    mesh=_mesh,
    scratch_types=[
        pltpu.VMEM((NGRP, EPB * L), jnp.int32),
        pltpu.VMEM((NBUF, EPB * L, EMB), jnp.bfloat16),
        pltpu.VMEM((BPW, EMB), jnp.float32),
    ] + [pltpu.SemaphoreType.DMA] * NBUF,
    compiler_params=pltpu.CompilerParams(
        use_tc_tiling_on_sc=False, needs_layout_passes=False),
)
def _lookup(ids_hbm, table_hbm, out_hbm, idx_v, rows_v, out_v, *sems):
    wid = lax.axis_index("s") * 2 + lax.axis_index("c")
    pltpu.sync_copy(ids_hbm.at[wid], idx_v)

    def _fire(g, b):
        return pltpu.async_copy(table_hbm.at[idx_v.at[g]], rows_v.at[b], sems[b])

    def _wait(g, b):
        pltpu.make_async_copy(table_hbm.at[idx_v.at[g]], rows_v.at[b], sems[b]).wait()

    for b in range(NBUF):
        _fire(b, b)

    def body(gg, carry):
        for b in range(NBUF):
            g = gg * NBUF + b
            _wait(g, b)
            for e in range(EPB):
                a0, a1 = plsc.unpack(
                    rows_v[b, e * L], format=plsc.PackFormat.INTERLEAVED)
                for j in range(1, L):
                    r0, r1 = plsc.unpack(
                        rows_v[b, e * L + j], format=plsc.PackFormat.INTERLEAVED)
                    a0 = a0 + r0
                    a1 = a1 + r1
                out_v[g * EPB + e, pl.ds(0, HALF)] = a0 * (1.0 / L)
                out_v[g * EPB + e, pl.ds(HALF, HALF)] = a1 * (1.0 / L)
            nxt = g + NBUF

            @pl.when(nxt < NGRP)
            def _():
                _fire(nxt, b)
        return carry

    lax.fori_loop(0, NGRP // NBUF, body, 0)
    pltpu.sync_copy(out_v, out_hbm.at[pl.ds(wid * BPW, BPW)])


def kernel(ids, table):
    out = _lookup(ids.reshape(NW, NGRP, EPB * L), table.astype(jnp.bfloat16))
    return jnp.stack([out[:, :HALF], out[:, HALF:]], axis=-1).reshape(B, EMB)


# speedup vs baseline: 1.0331x; 1.0331x over previous
import functools

import jax
import jax.numpy as jnp
from jax import lax
from jax.experimental import pallas as pl
from jax.experimental.pallas import tpu as pltpu
from jax.experimental.pallas import tpu_sc as plsc

B = 16384
L = 50
EMB = 32
NW = 32
BPW = B // NW
NBUF = 8
HALF = 16

_mesh = plsc.VectorSubcoreMesh(core_axis_name="c", subcore_axis_name="s")


@functools.partial(
    pl.kernel,
    out_type=jax.ShapeDtypeStruct((B, EMB), jnp.float32),
    mesh=_mesh,
    scratch_types=[
        pltpu.VMEM((BPW, L), jnp.int32),
        pltpu.VMEM((NBUF, L, EMB), jnp.bfloat16),
        pltpu.VMEM((BPW, EMB), jnp.float32),
    ] + [pltpu.SemaphoreType.DMA] * NBUF,
    compiler_params=pltpu.CompilerParams(
        use_tc_tiling_on_sc=False, needs_layout_passes=False),
)
def _lookup(ids_hbm, table_hbm, out_hbm, idx_v, rows_v, out_v, *sems):
    wid = lax.axis_index("s") * 2 + lax.axis_index("c")
    pltpu.sync_copy(ids_hbm.at[wid], idx_v)

    def _fire(e, b):
        return pltpu.async_copy(table_hbm.at[idx_v.at[e]], rows_v.at[b], sems[b])

    def _wait(e, b):
        pltpu.make_async_copy(table_hbm.at[idx_v.at[e]], rows_v.at[b], sems[b]).wait()

    for b in range(NBUF):
        _fire(b, b)

    def body(gg, carry):
        for b in range(NBUF):
            e = gg * NBUF + b
            _wait(e, b)
            a0, a1 = plsc.unpack(rows_v[b, 0], format=plsc.PackFormat.INTERLEAVED)
            for j in range(1, L):
                r0, r1 = plsc.unpack(
                    rows_v[b, j], format=plsc.PackFormat.INTERLEAVED)
                a0 = a0 + r0
                a1 = a1 + r1
            out_v[e, pl.ds(0, HALF)] = a0 * (1.0 / L)
            out_v[e, pl.ds(HALF, HALF)] = a1 * (1.0 / L)
            nxt = e + NBUF

            @pl.when(nxt < BPW)
            def _():
                _fire(nxt, b)
        return carry

    lax.fori_loop(0, BPW // NBUF, body, 0)
    pltpu.sync_copy(out_v, out_hbm.at[pl.ds(wid * BPW, BPW)])


def kernel(ids, table):
    out = _lookup(ids.reshape(NW, BPW, L), table.astype(jnp.bfloat16))
    return jnp.stack([out[:, :HALF], out[:, HALF:]], axis=-1).reshape(B, EMB)


# speedup vs baseline: 1.5014x; 1.4534x over previous
import functools

import jax
import jax.numpy as jnp
from jax import lax
from jax.experimental import pallas as pl
from jax.experimental.pallas import tpu as pltpu
from jax.experimental.pallas import tpu_sc as plsc

B = 16384
L = 50
EMB = 32
NW = 32
BPW = B // NW
NBUF = 8
HALF = 16

_mesh = plsc.VectorSubcoreMesh(core_axis_name="c", subcore_axis_name="s")


@functools.partial(
    pl.kernel,
    out_type=jax.ShapeDtypeStruct((B, EMB), jnp.float32),
    mesh=_mesh,
    scratch_types=[
        pltpu.VMEM((BPW, L), jnp.int32),
        pltpu.VMEM((NBUF, L, EMB), jnp.float32),
        pltpu.VMEM((BPW, EMB), jnp.float32),
    ] + [pltpu.SemaphoreType.DMA] * NBUF,
    compiler_params=pltpu.CompilerParams(
        use_tc_tiling_on_sc=False, needs_layout_passes=False),
)
def _lookup(ids_hbm, table_hbm, out_hbm, idx_v, rows_v, out_v, *sems):
    wid = lax.axis_index("s") * 2 + lax.axis_index("c")
    pltpu.sync_copy(ids_hbm.at[wid], idx_v)

    def _fire(e, b):
        return pltpu.async_copy(table_hbm.at[idx_v.at[e]], rows_v.at[b], sems[b])

    def _wait(e, b):
        pltpu.make_async_copy(table_hbm.at[idx_v.at[e]], rows_v.at[b], sems[b]).wait()

    for b in range(NBUF):
        _fire(b, b)

    def body(gg, carry):
        for b in range(NBUF):
            e = gg * NBUF + b
            _wait(e, b)
            a0 = rows_v[b, 0, pl.ds(0, HALF)]
            a1 = rows_v[b, 0, pl.ds(HALF, HALF)]
            for j in range(1, L):
                a0 = a0 + rows_v[b, j, pl.ds(0, HALF)]
                a1 = a1 + rows_v[b, j, pl.ds(HALF, HALF)]
            out_v[e, pl.ds(0, HALF)] = a0 * (1.0 / L)
            out_v[e, pl.ds(HALF, HALF)] = a1 * (1.0 / L)
            nxt = e + NBUF

            @pl.when(nxt < BPW)
            def _():
                _fire(nxt, b)
        return carry

    lax.fori_loop(0, BPW // NBUF, body, 0)
    pltpu.sync_copy(out_v, out_hbm.at[pl.ds(wid * BPW, BPW)])


def kernel(ids, table):
    return _lookup(ids.reshape(NW, BPW, L), table)
